# baseline (speedup 1.0000x reference)
import functools

import jax
import jax.numpy as jnp
from jax import lax
from jax.experimental import pallas as pl
from jax.experimental.pallas import tpu as pltpu
from jax.experimental.pallas import tpu_sc as plsc

N = 10000
E = 320000
D = 128
G = 64

NPAD = 10240
PAD_IDX = N
RNG = 320
ACC_ROWS = RNG + 8
EPT = E // 32
ECHUNK = 2000
NCH = EPT // ECHUNK
CAPP = (EPT // 128 + 6) * 128
BSTRIDE = 160
BLK = 512
NBLK = NPAD // BLK

_mesh = plsc.VectorSubcoreMesh(core_axis_name="c", subcore_axis_name="s")


@functools.partial(
    pl.kernel,
    out_type=[jax.ShapeDtypeStruct((32 * 32 * CAPP,), jnp.int32),
              jax.ShapeDtypeStruct((32 * 32 * CAPP,), jnp.int32),
              jax.ShapeDtypeStruct((32 * 512,), jnp.int32)],
    mesh=_mesh,
    scratch_types=[
        pltpu.VMEM((ECHUNK,), jnp.int32),
        pltpu.VMEM((ECHUNK,), jnp.int32),
        pltpu.VMEM((32 * BSTRIDE,), jnp.int32),
        pltpu.VMEM((32 * BSTRIDE,), jnp.int32),
        pltpu.VMEM((512,), jnp.int32),
        pltpu.SMEM((64,), jnp.int32),
    ],
)
def _part_sc(src_hbm, dst_hbm, zeros_hbm, selsrc_hbm, seldst_hbm, cnt_hbm,
             srcc_v, dstc_v, bsrc_v, bdst_v, cstage_v, sm):
    cc = lax.axis_index("c")
    ss = lax.axis_index("s")
    t = cc * 16 + ss
    iota = lax.iota(jnp.int32, 16)
    z16 = jnp.zeros((16,), jnp.int32)
    pltpu.sync_copy(zeros_hbm, bsrc_v)
    pltpu.sync_copy(zeros_hbm, bdst_v)
    for q in range(32):
        sm[q] = 0
        sm[32 + q] = 0

    def chunk_body(tt, carry):
        e0 = t * EPT + tt * ECHUNK
        pltpu.sync_copy(src_hbm.at[pl.ds(e0, ECHUNK)], srcc_v)
        pltpu.sync_copy(dst_hbm.at[pl.ds(e0, ECHUNK)], dstc_v)

        def vec_body(i, carry2):
            dvec = dstc_v[pl.ds(i * 16, 16)]
            svec = srcc_v[pl.ds(i * 16, 16)]
            qvec = (dvec * 6554) >> 21
            dlvec = dvec - qvec * RNG
            for l in range(16):
                q = qvec[l]
                s_ = svec[l]
                dl = dlvec[l]
                w = sm[q]
                lane = w & 15
                base = (w - lane) + q * BSTRIDE
                oh = iota == lane
                plsc.addupdate(bsrc_v.at[pl.ds(base, 16)],
                               jnp.where(oh, s_, 0))
                plsc.addupdate(bdst_v.at[pl.ds(base, 16)],
                               jnp.where(oh, dl, 0))
                w2 = w + 1

                @pl.when(w2 == 128)
                def _flush():
                    nb = sm[32 + q]
                    off = (q * 32 + t) * CAPP + nb * 128
                    pltpu.sync_copy(bsrc_v.at[pl.ds(q * BSTRIDE, 128)],
                                    selsrc_hbm.at[pl.ds(off, 128)])
                    pltpu.sync_copy(bdst_v.at[pl.ds(q * BSTRIDE, 128)],
                                    seldst_hbm.at[pl.ds(off, 128)])
                    for ii in range(8):
                        bsrc_v[pl.ds(q * BSTRIDE + ii * 16, 16)] = z16
                        bdst_v[pl.ds(q * BSTRIDE + ii * 16, 16)] = z16
                    sm[32 + q] = nb + 1

                sm[q] = w2 & 127
            return carry2

        lax.fori_loop(0, ECHUNK // 16, vec_body, 0)
        return carry

    lax.fori_loop(0, NCH, chunk_body, 0)

    for q in range(32):
        w = sm[q]
        nb = sm[32 + q]
        for ii in range(8):
            pos = iota + (ii * 16)
            sv = bsrc_v[pl.ds(q * BSTRIDE + ii * 16, 16)]
            dv = bdst_v[pl.ds(q * BSTRIDE + ii * 16, 16)]
            bsrc_v[pl.ds(q * BSTRIDE + ii * 16, 16)] = \
                jnp.where(pos >= w, PAD_IDX, sv)
            bdst_v[pl.ds(q * BSTRIDE + ii * 16, 16)] = \
                jnp.where(pos >= w, RNG, dv)
        off = (q * 32 + t) * CAPP + nb * 128
        pltpu.sync_copy(bsrc_v.at[pl.ds(q * BSTRIDE, 128)],
                        selsrc_hbm.at[pl.ds(off, 128)])
        pltpu.sync_copy(bdst_v.at[pl.ds(q * BSTRIDE, 128)],
                        seldst_hbm.at[pl.ds(off, 128)])
        for ii in range(8):
            bsrc_v[pl.ds(q * BSTRIDE + ii * 16, 16)] = z16 + PAD_IDX
            bdst_v[pl.ds(q * BSTRIDE + ii * 16, 16)] = z16 + RNG
        for jj in range(3):
            offj = off + (jj + 1) * 128
            pltpu.sync_copy(bsrc_v.at[pl.ds(q * BSTRIDE, 128)],
                            selsrc_hbm.at[pl.ds(offj, 128)])
            pltpu.sync_copy(bdst_v.at[pl.ds(q * BSTRIDE, 128)],
                            seldst_hbm.at[pl.ds(offj, 128)])
        cstage_v[pl.ds(q * 16, 16)] = z16 + (nb * 128 + w)
    pltpu.sync_copy(cstage_v, cnt_hbm.at[pl.ds(t * 512, 512)])


@functools.partial(
    pl.kernel,
    out_type=jax.ShapeDtypeStruct((NPAD * 16,), jnp.float32),
    mesh=_mesh,
    scratch_types=[
        pltpu.VMEM((128,), jnp.int32),
        pltpu.VMEM((16,), jnp.int32),
        pltpu.VMEM((ACC_ROWS * 16,), jnp.float32),
    ],
)
def _deg_sc(seldst_hbm, cnt_hbm, zeros_hbm, dp_hbm, dstl_v, cnt_v, acc_v):
    cc = lax.axis_index("c")
    ss = lax.axis_index("s")
    b = cc * 16 + ss
    pltpu.sync_copy(zeros_hbm, acc_v)
    ones = jnp.ones((16,), jnp.float32)

    def src_body(s, carry):
        pltpu.sync_copy(cnt_hbm.at[pl.ds(s * 512 + b * 16, 16)], cnt_v)
        cvec = cnt_v[pl.ds(0, 16)]
        nblk = (cvec[0] + 127) >> 7

        def blk_body(k, carry2):
            off = (b * 32 + s) * CAPP + k * 128
            pltpu.sync_copy(seldst_hbm.at[pl.ds(off, 128)], dstl_v)

            def grp_body(g, carry3):
                dvec = dstl_v[pl.ds(g * 16, 16)]
                for l in range(16):
                    d = dvec[l]
                    plsc.addupdate(acc_v.at[pl.ds(d * 16, 16)], ones)
                return carry3

            lax.fori_loop(0, 8, grp_body, 0)
            return carry2

        lax.fori_loop(0, nblk, blk_body, 0)
        return carry

    lax.fori_loop(0, 32, src_body, 0)
    pltpu.sync_copy(acc_v.at[pl.ds(0, RNG * 16)],
                    dp_hbm.at[pl.ds(b * RNG * 16, RNG * 16)])


@functools.partial(
    pl.kernel,
    out_type=jax.ShapeDtypeStruct((NPAD * D,), jnp.float32),
    mesh=_mesh,
    scratch_types=[
        pltpu.VMEM((512,), jnp.int32),
        pltpu.VMEM((512,), jnp.int32),
        pltpu.VMEM((16,), jnp.int32),
        pltpu.VMEM((4, 128, D), jnp.float32),
        pltpu.VMEM((ACC_ROWS * D,), jnp.float32),
        pltpu.SemaphoreType.DMA,
        pltpu.SemaphoreType.DMA,
        pltpu.SemaphoreType.DMA,
        pltpu.SemaphoreType.DMA,
    ],
)
def _agg_sc(hs_hbm, selsrc_hbm, seldst_hbm, cnt_hbm, zeros_hbm, out_hbm,
            src_v, dstl_v, cnt_v, rows_v, acc_v, sem0, sem1, sem2, sem3):
    cc = lax.axis_index("c")
    ss = lax.axis_index("s")
    b = cc * 16 + ss
    sems = (sem0, sem1, sem2, sem3)
    pltpu.sync_copy(zeros_hbm, acc_v)

    def src_body(s, carry):
        pltpu.sync_copy(cnt_hbm.at[pl.ds(s * 512 + b * 16, 16)], cnt_v)
        cvec = cnt_v[pl.ds(0, 16)]
        nsb = (cvec[0] + 511) >> 9

        def sb_body(p, carry2):
            off = (b * 32 + s) * CAPP + p * 512
            pltpu.sync_copy(selsrc_hbm.at[pl.ds(off, 512)], src_v)
            pltpu.sync_copy(seldst_hbm.at[pl.ds(off, 512)], dstl_v)
            cps = [pltpu.async_copy(hs_hbm.at[src_v.at[pl.ds(j * 128, 128)]],
                                    rows_v.at[j], sems[j])
                   for j in range(4)]
            for j in range(4):
                cps[j].wait()

                def grp_body(g, carry3, j=j):
                    dvec = dstl_v[pl.ds(j * 128 + g * 16, 16)]
                    for l in range(16):
                        d = dvec[l]
                        base = d * D
                        for kk in range(8):
                            val = rows_v[j, g * 16 + l, pl.ds(kk * 16, 16)]
                            plsc.addupdate(
                                acc_v.at[pl.ds(base + kk * 16, 16)], val)
                    return carry3

                lax.fori_loop(0, 8, grp_body, 0)
            return carry2

        lax.fori_loop(0, nsb, sb_body, 0)
        return carry

    lax.fori_loop(0, 32, src_body, 0)
    pltpu.sync_copy(acc_v.at[pl.ds(0, RNG * D)],
                    out_hbm.at[pl.ds(b * RNG * D, RNG * D)])


def _dinv_block(dp_ref):
    deg = jnp.sum(dp_ref[...], axis=1, keepdims=True) * 0.0625 + 1.0
    return lax.rsqrt(deg)


def _tc1_body(x_ref, w_ref, dp_ref, hs_ref):
    dinv = _dinv_block(dp_ref)
    xw = jnp.dot(x_ref[...], w_ref[...], preferred_element_type=jnp.float32)
    hs_ref[...] = xw * dinv


def _tc2_body(hs_ref, p_ref, dp_ref, b_ref, w_ref, out_ref):
    dinv = _dinv_block(dp_ref)
    tot = hs_ref[...] + p_ref[...]
    h = jnp.maximum(tot * dinv + b_ref[...], 0.0)
    out_ref[...] = (
        jnp.dot(h, w_ref[...], preferred_element_type=jnp.float32) * dinv
    )


def _tc3_body(hs_ref, p_ref, dp_ref, b_ref, batch_ref, sums_ref, cnts_ref):
    i = pl.program_id(0)
    dinv = _dinv_block(dp_ref)
    h3 = (hs_ref[...] + p_ref[...]) * dinv + b_ref[...]
    gids = lax.broadcasted_iota(jnp.int32, (1, G), 1)
    onehot = (batch_ref[...] == gids).astype(jnp.float32)
    dn = (((0,), (0,)), ((), ()))
    sm = lax.dot_general(onehot, h3, dn, preferred_element_type=jnp.float32)
    ones = jnp.ones((BLK, D), jnp.float32)
    cn = lax.dot_general(onehot, ones, dn, preferred_element_type=jnp.float32)

    @pl.when(i == 0)
    def _init():
        sums_ref[...] = sm
        cnts_ref[...] = cn

    @pl.when(i > 0)
    def _acc():
        sums_ref[...] += sm
        cnts_ref[...] += cn


def _tc4_body(sums_ref, cnts_ref, w1, b1, w2, b2, w3, b3, w4, b4, f_ref,
              y_ref):
    f = sums_ref[...] / jnp.maximum(cnts_ref[...], 1.0)
    f_ref[...] = f
    y = jnp.maximum(jnp.dot(f, w1[...], preferred_element_type=jnp.float32)
                    + b1[...], 0.0)
    y = jnp.maximum(jnp.dot(y, w2[...], preferred_element_type=jnp.float32)
                    + b2[...], 0.0)
    y = jnp.maximum(jnp.dot(y, w3[...], preferred_element_type=jnp.float32)
                    + b3[...], 0.0)
    y_ref[...] = jnp.dot(y, w4[...], preferred_element_type=jnp.float32) \
        + b4[...]


def _full(shape):
    return pl.BlockSpec(shape, lambda i: (0,) * len(shape))


_BS_ROWS = pl.BlockSpec((BLK, D), lambda i: (i, 0))
_BS_DP = pl.BlockSpec((BLK, 16), lambda i: (i, 0))


def _tc1(x_pad, W1, dp):
    return pl.pallas_call(
        _tc1_body,
        grid=(NBLK,),
        in_specs=[_BS_ROWS, _full((D, D)), _BS_DP],
        out_specs=_BS_ROWS,
        out_shape=jax.ShapeDtypeStruct((NPAD, D), jnp.float32),
    )(x_pad, W1, dp)


def _tc2(hs, p, dp, b, Wn):
    return pl.pallas_call(
        _tc2_body,
        grid=(NBLK,),
        in_specs=[_BS_ROWS, _BS_ROWS, _BS_DP, _full((1, D)), _full((D, D))],
        out_specs=_BS_ROWS,
        out_shape=jax.ShapeDtypeStruct((NPAD, D), jnp.float32),
    )(hs, p, dp, b, Wn)


def _tc3(hs, p, dp, b, batch_pad):
    return pl.pallas_call(
        _tc3_body,
        grid=(NBLK,),
        in_specs=[_BS_ROWS, _BS_ROWS, _BS_DP, _full((1, D)),
                  pl.BlockSpec((BLK, 1), lambda i: (i, 0))],
        out_specs=[_full((G, D)), _full((G, D))],
        out_shape=[jax.ShapeDtypeStruct((G, D), jnp.float32),
                   jax.ShapeDtypeStruct((G, D), jnp.float32)],
    )(hs, p, dp, b, batch_pad)


def _tc4(sums, cnts, fcW1, fcb1, fcW2, fcb2, fcW3, fcb3, fcW4, fcb4):
    return pl.pallas_call(
        _tc4_body,
        out_shape=[jax.ShapeDtypeStruct((G, D), jnp.float32),
                   jax.ShapeDtypeStruct((G, 32), jnp.float32)],
    )(sums, cnts, fcW1, fcb1, fcW2, fcb2, fcW3, fcb3, fcW4, fcb4)


def kernel(x, edge_index, batch, W1, b1, W2, b2, W3, b3, fcW1, fcb1, fcW2,
           fcb2, fcW3, fcb3, fcW4, fcb4):
    x_pad = jnp.zeros((NPAD, D), jnp.float32).at[:N].set(x)
    src_flat = edge_index[0]
    dst_flat = edge_index[1]
    batch_pad = jnp.concatenate(
        [batch, jnp.full((NPAD - N,), G, jnp.int32)]).reshape(NPAD, 1)
    zeros_i = jnp.zeros((32 * BSTRIDE,), jnp.int32)
    zeros16 = jnp.zeros((ACC_ROWS * 16,), jnp.float32)
    zerosD = jnp.zeros((ACC_ROWS * D,), jnp.float32)

    selsrc, seldst, cnt = _part_sc(src_flat, dst_flat, zeros_i)
    dp = _deg_sc(seldst, cnt, zeros16).reshape(NPAD, 16)

    hs1 = _tc1(x_pad, W1, dp)
    p1 = _agg_sc(hs1, selsrc, seldst, cnt, zerosD).reshape(NPAD, D)
    hs2 = _tc2(hs1, p1, dp, b1.reshape(1, D), W2)
    p2 = _agg_sc(hs2, selsrc, seldst, cnt, zerosD).reshape(NPAD, D)
    hs3 = _tc2(hs2, p2, dp, b2.reshape(1, D), W3)
    p3 = _agg_sc(hs3, selsrc, seldst, cnt, zerosD).reshape(NPAD, D)
    sums, cnts = _tc3(hs3, p3, dp, b3.reshape(1, D), batch_pad)
    f, y = _tc4(sums, cnts, fcW1, fcb1.reshape(1, 4 * D), fcW2,
                fcb2.reshape(1, 2 * D), fcW3, fcb3.reshape(1, D), fcW4,
                fcb4.reshape(1, 32))
    return (f, y)


# speedup vs baseline: 3.0790x; 3.0790x over previous
import functools

import jax
import jax.numpy as jnp
from jax import lax
from jax.experimental import pallas as pl
from jax.experimental.pallas import tpu as pltpu
from jax.experimental.pallas import tpu_sc as plsc

N = 10000
E = 320000
D = 128
G = 64

NPAD = 10240
PAD_IDX = N
RNG = 320
ACC_ROWS = RNG + 8
EPT = E // 32
ECHUNK = 2000
NCH = EPT // ECHUNK
CAPP = (EPT // 128 + 6) * 128
BSTRIDE = 160
BLK = 512
NBLK = NPAD // BLK

_mesh = plsc.VectorSubcoreMesh(core_axis_name="c", subcore_axis_name="s")


@functools.partial(
    pl.kernel,
    out_type=[jax.ShapeDtypeStruct((32 * 32 * CAPP,), jnp.int32),
              jax.ShapeDtypeStruct((32 * 32 * CAPP,), jnp.int32),
              jax.ShapeDtypeStruct((32 * 512,), jnp.int32)],
    mesh=_mesh,
    scratch_types=[
        pltpu.VMEM((ECHUNK,), jnp.int32),
        pltpu.VMEM((ECHUNK,), jnp.int32),
        pltpu.VMEM((32 * BSTRIDE,), jnp.int32),
        pltpu.VMEM((32 * BSTRIDE,), jnp.int32),
        pltpu.VMEM((512,), jnp.int32),
        pltpu.SMEM((64,), jnp.int32),
    ],
)
def _part_sc(src_hbm, dst_hbm, zeros_hbm, selsrc_hbm, seldst_hbm, cnt_hbm,
             srcc_v, dstc_v, bsrc_v, bdst_v, cstage_v, sm):
    cc = lax.axis_index("c")
    ss = lax.axis_index("s")
    t = cc * 16 + ss
    iota = lax.iota(jnp.int32, 16)
    z16 = jnp.zeros((16,), jnp.int32)
    pltpu.sync_copy(zeros_hbm, bsrc_v)
    pltpu.sync_copy(zeros_hbm, bdst_v)
    for q in range(32):
        sm[q] = 0
        sm[32 + q] = 0

    def chunk_body(tt, carry):
        e0 = t * EPT + tt * ECHUNK
        pltpu.sync_copy(src_hbm.at[pl.ds(e0, ECHUNK)], srcc_v)
        pltpu.sync_copy(dst_hbm.at[pl.ds(e0, ECHUNK)], dstc_v)

        def vec_body(i, carry2):
            dvec = dstc_v[pl.ds(i * 16, 16)]
            svec = srcc_v[pl.ds(i * 16, 16)]
            qvec = (dvec * 6554) >> 21
            dlvec = dvec - qvec * RNG
            for l in range(16):
                q = qvec[l]
                s_ = svec[l]
                dl = dlvec[l]
                w = sm[q]
                lane = w & 15
                base = (w - lane) + q * BSTRIDE
                oh = iota == lane
                plsc.addupdate(bsrc_v.at[pl.ds(base, 16)],
                               jnp.where(oh, s_, 0))
                plsc.addupdate(bdst_v.at[pl.ds(base, 16)],
                               jnp.where(oh, dl, 0))
                w2 = w + 1

                @pl.when(w2 == 128)
                def _flush():
                    nb = sm[32 + q]
                    off = (q * 32 + t) * CAPP + nb * 128
                    pltpu.sync_copy(bsrc_v.at[pl.ds(q * BSTRIDE, 128)],
                                    selsrc_hbm.at[pl.ds(off, 128)])
                    pltpu.sync_copy(bdst_v.at[pl.ds(q * BSTRIDE, 128)],
                                    seldst_hbm.at[pl.ds(off, 128)])
                    for ii in range(8):
                        bsrc_v[pl.ds(q * BSTRIDE + ii * 16, 16)] = z16
                        bdst_v[pl.ds(q * BSTRIDE + ii * 16, 16)] = z16
                    sm[32 + q] = nb + 1

                sm[q] = w2 & 127
            return carry2

        lax.fori_loop(0, ECHUNK // 16, vec_body, 0)
        return carry

    lax.fori_loop(0, NCH, chunk_body, 0)

    for q in range(32):
        w = sm[q]
        nb = sm[32 + q]
        for ii in range(8):
            pos = iota + (ii * 16)
            sv = bsrc_v[pl.ds(q * BSTRIDE + ii * 16, 16)]
            dv = bdst_v[pl.ds(q * BSTRIDE + ii * 16, 16)]
            bsrc_v[pl.ds(q * BSTRIDE + ii * 16, 16)] = \
                jnp.where(pos >= w, PAD_IDX, sv)
            bdst_v[pl.ds(q * BSTRIDE + ii * 16, 16)] = \
                jnp.where(pos >= w, RNG, dv)
        off = (q * 32 + t) * CAPP + nb * 128
        pltpu.sync_copy(bsrc_v.at[pl.ds(q * BSTRIDE, 128)],
                        selsrc_hbm.at[pl.ds(off, 128)])
        pltpu.sync_copy(bdst_v.at[pl.ds(q * BSTRIDE, 128)],
                        seldst_hbm.at[pl.ds(off, 128)])
        for ii in range(8):
            bsrc_v[pl.ds(q * BSTRIDE + ii * 16, 16)] = z16 + PAD_IDX
            bdst_v[pl.ds(q * BSTRIDE + ii * 16, 16)] = z16 + RNG
        for jj in range(3):
            offj = off + (jj + 1) * 128
            pltpu.sync_copy(bsrc_v.at[pl.ds(q * BSTRIDE, 128)],
                            selsrc_hbm.at[pl.ds(offj, 128)])
            pltpu.sync_copy(bdst_v.at[pl.ds(q * BSTRIDE, 128)],
                            seldst_hbm.at[pl.ds(offj, 128)])
        cstage_v[pl.ds(q * 16, 16)] = z16 + (nb * 128 + w)
    pltpu.sync_copy(cstage_v, cnt_hbm.at[pl.ds(t * 512, 512)])


@functools.partial(
    pl.kernel,
    out_type=jax.ShapeDtypeStruct((NPAD * 16,), jnp.float32),
    mesh=_mesh,
    scratch_types=[
        pltpu.VMEM((128,), jnp.int32),
        pltpu.VMEM((16,), jnp.int32),
        pltpu.VMEM((ACC_ROWS * 16,), jnp.float32),
    ],
)
def _deg_sc(seldst_hbm, cnt_hbm, zeros_hbm, dp_hbm, dstl_v, cnt_v, acc_v):
    cc = lax.axis_index("c")
    ss = lax.axis_index("s")
    b = cc * 16 + ss
    pltpu.sync_copy(zeros_hbm, acc_v)
    ones = jnp.ones((16,), jnp.float32)

    def src_body(s, carry):
        pltpu.sync_copy(cnt_hbm.at[pl.ds(s * 512 + b * 16, 16)], cnt_v)
        cvec = cnt_v[pl.ds(0, 16)]
        nblk = (cvec[0] + 127) >> 7

        def blk_body(k, carry2):
            off = (b * 32 + s) * CAPP + k * 128
            pltpu.sync_copy(seldst_hbm.at[pl.ds(off, 128)], dstl_v)

            def grp_body(g, carry3):
                dvec = dstl_v[pl.ds(g * 16, 16)]
                for l in range(16):
                    d = dvec[l]
                    plsc.addupdate(acc_v.at[pl.ds(d * 16, 16)], ones)
                return carry3

            lax.fori_loop(0, 8, grp_body, 0)
            return carry2

        lax.fori_loop(0, nblk, blk_body, 0)
        return carry

    lax.fori_loop(0, 32, src_body, 0)
    pltpu.sync_copy(acc_v.at[pl.ds(0, RNG * 16)],
                    dp_hbm.at[pl.ds(b * RNG * 16, RNG * 16)])


@functools.partial(
    pl.kernel,
    out_type=jax.ShapeDtypeStruct((NPAD * D,), jnp.float32),
    mesh=_mesh,
    scratch_types=[
        pltpu.VMEM((128,), jnp.int32),
        pltpu.VMEM((128,), jnp.int32),
        pltpu.VMEM((16,), jnp.int32),
        pltpu.VMEM((128, D), jnp.float32),
        pltpu.VMEM((ACC_ROWS * D,), jnp.float32),
        pltpu.SemaphoreType.DMA,
    ],
)
def _agg_sc(hs_hbm, selsrc_hbm, seldst_hbm, cnt_hbm, zeros_hbm, out_hbm,
            src_v, dstl_v, cnt_v, rows_v, acc_v, sem):
    cc = lax.axis_index("c")
    ss = lax.axis_index("s")
    b = cc * 16 + ss
    pltpu.sync_copy(zeros_hbm, acc_v)

    def src_body(s, carry):
        pltpu.sync_copy(cnt_hbm.at[pl.ds(s * 512 + b * 16, 16)], cnt_v)
        cvec = cnt_v[pl.ds(0, 16)]
        nblk = (cvec[0] + 127) >> 7

        def blk_body(k, carry2):
            off = (b * 32 + s) * CAPP + k * 128
            pltpu.sync_copy(selsrc_hbm.at[pl.ds(off, 128)], src_v)
            pltpu.sync_copy(seldst_hbm.at[pl.ds(off, 128)], dstl_v)
            pltpu.async_copy(hs_hbm.at[src_v], rows_v, sem).wait()

            def grp_body(g, carry3):
                dvec = dstl_v[pl.ds(g * 16, 16)]
                for l in range(16):
                    d = dvec[l]
                    base = d * D
                    for kk in range(8):
                        val = rows_v[g * 16 + l, pl.ds(kk * 16, 16)]
                        plsc.addupdate(acc_v.at[pl.ds(base + kk * 16, 16)],
                                       val)
                return carry3

            lax.fori_loop(0, 8, grp_body, 0)
            return carry2

        lax.fori_loop(0, nblk, blk_body, 0)
        return carry

    lax.fori_loop(0, 32, src_body, 0)
    pltpu.sync_copy(acc_v.at[pl.ds(0, RNG * D)],
                    out_hbm.at[pl.ds(b * RNG * D, RNG * D)])


def _dinv_block(dp_ref):
    deg = jnp.sum(dp_ref[...], axis=1, keepdims=True) * 0.0625 + 1.0
    return lax.rsqrt(deg)


def _tc1_body(x_ref, w_ref, dp_ref, hs_ref):
    dinv = _dinv_block(dp_ref)
    xw = jnp.dot(x_ref[...], w_ref[...], preferred_element_type=jnp.float32)
    hs_ref[...] = xw * dinv


def _tc2_body(hs_ref, p_ref, dp_ref, b_ref, w_ref, out_ref):
    dinv = _dinv_block(dp_ref)
    tot = hs_ref[...] + p_ref[...]
    h = jnp.maximum(tot * dinv + b_ref[...], 0.0)
    out_ref[...] = (
        jnp.dot(h, w_ref[...], preferred_element_type=jnp.float32) * dinv
    )


def _tc3_body(hs_ref, p_ref, dp_ref, b_ref, batch_ref, sums_ref, cnts_ref):
    i = pl.program_id(0)
    dinv = _dinv_block(dp_ref)
    h3 = (hs_ref[...] + p_ref[...]) * dinv + b_ref[...]
    gids = lax.broadcasted_iota(jnp.int32, (1, G), 1)
    onehot = (batch_ref[...] == gids).astype(jnp.float32)
    dn = (((0,), (0,)), ((), ()))
    sm = lax.dot_general(onehot, h3, dn, preferred_element_type=jnp.float32)
    ones = jnp.ones((BLK, D), jnp.float32)
    cn = lax.dot_general(onehot, ones, dn, preferred_element_type=jnp.float32)

    @pl.when(i == 0)
    def _init():
        sums_ref[...] = sm
        cnts_ref[...] = cn

    @pl.when(i > 0)
    def _acc():
        sums_ref[...] += sm
        cnts_ref[...] += cn


def _tc4_body(sums_ref, cnts_ref, w1, b1, w2, b2, w3, b3, w4, b4, f_ref,
              y_ref):
    f = sums_ref[...] / jnp.maximum(cnts_ref[...], 1.0)
    f_ref[...] = f
    y = jnp.maximum(jnp.dot(f, w1[...], preferred_element_type=jnp.float32)
                    + b1[...], 0.0)
    y = jnp.maximum(jnp.dot(y, w2[...], preferred_element_type=jnp.float32)
                    + b2[...], 0.0)
    y = jnp.maximum(jnp.dot(y, w3[...], preferred_element_type=jnp.float32)
                    + b3[...], 0.0)
    y_ref[...] = jnp.dot(y, w4[...], preferred_element_type=jnp.float32) \
        + b4[...]


def _full(shape):
    return pl.BlockSpec(shape, lambda i: (0,) * len(shape))


_BS_ROWS = pl.BlockSpec((BLK, D), lambda i: (i, 0))
_BS_DP = pl.BlockSpec((BLK, 16), lambda i: (i, 0))


def _tc1(x_pad, W1, dp):
    return pl.pallas_call(
        _tc1_body,
        grid=(NBLK,),
        in_specs=[_BS_ROWS, _full((D, D)), _BS_DP],
        out_specs=_BS_ROWS,
        out_shape=jax.ShapeDtypeStruct((NPAD, D), jnp.float32),
    )(x_pad, W1, dp)


def _tc2(hs, p, dp, b, Wn):
    return pl.pallas_call(
        _tc2_body,
        grid=(NBLK,),
        in_specs=[_BS_ROWS, _BS_ROWS, _BS_DP, _full((1, D)), _full((D, D))],
        out_specs=_BS_ROWS,
        out_shape=jax.ShapeDtypeStruct((NPAD, D), jnp.float32),
    )(hs, p, dp, b, Wn)


def _tc3(hs, p, dp, b, batch_pad):
    return pl.pallas_call(
        _tc3_body,
        grid=(NBLK,),
        in_specs=[_BS_ROWS, _BS_ROWS, _BS_DP, _full((1, D)),
                  pl.BlockSpec((BLK, 1), lambda i: (i, 0))],
        out_specs=[_full((G, D)), _full((G, D))],
        out_shape=[jax.ShapeDtypeStruct((G, D), jnp.float32),
                   jax.ShapeDtypeStruct((G, D), jnp.float32)],
    )(hs, p, dp, b, batch_pad)


def _tc4(sums, cnts, fcW1, fcb1, fcW2, fcb2, fcW3, fcb3, fcW4, fcb4):
    return pl.pallas_call(
        _tc4_body,
        out_shape=[jax.ShapeDtypeStruct((G, D), jnp.float32),
                   jax.ShapeDtypeStruct((G, 32), jnp.float32)],
    )(sums, cnts, fcW1, fcb1, fcW2, fcb2, fcW3, fcb3, fcW4, fcb4)


def kernel(x, edge_index, batch, W1, b1, W2, b2, W3, b3, fcW1, fcb1, fcW2,
           fcb2, fcW3, fcb3, fcW4, fcb4):
    x_pad = jnp.zeros((NPAD, D), jnp.float32).at[:N].set(x)
    src_flat = edge_index[0]
    dst_flat = edge_index[1]
    batch_pad = jnp.concatenate(
        [batch, jnp.full((NPAD - N,), G, jnp.int32)]).reshape(NPAD, 1)
    zeros_i = jnp.zeros((32 * BSTRIDE,), jnp.int32)
    zeros16 = jnp.zeros((ACC_ROWS * 16,), jnp.float32)
    zerosD = jnp.zeros((ACC_ROWS * D,), jnp.float32)

    selsrc, seldst, cnt = _part_sc(src_flat, dst_flat, zeros_i)
    dp = _deg_sc(seldst, cnt, zeros16).reshape(NPAD, 16)

    hs1 = _tc1(x_pad, W1, dp)
    p1 = _agg_sc(hs1, selsrc, seldst, cnt, zerosD).reshape(NPAD, D)
    hs2 = _tc2(hs1, p1, dp, b1.reshape(1, D), W2)
    p2 = _agg_sc(hs2, selsrc, seldst, cnt, zerosD).reshape(NPAD, D)
    hs3 = _tc2(hs2, p2, dp, b2.reshape(1, D), W3)
    p3 = _agg_sc(hs3, selsrc, seldst, cnt, zerosD).reshape(NPAD, D)
    sums, cnts = _tc3(hs3, p3, dp, b3.reshape(1, D), batch_pad)
    f, y = _tc4(sums, cnts, fcW1, fcb1.reshape(1, 4 * D), fcW2,
                fcb2.reshape(1, 2 * D), fcW3, fcb3.reshape(1, D), fcW4,
                fcb4.reshape(1, 32))
    return (f, y)


# speedup vs baseline: 3.0791x; 1.0000x over previous
import functools

import jax
import jax.numpy as jnp
from jax import lax
from jax.experimental import pallas as pl
from jax.experimental.pallas import tpu as pltpu
from jax.experimental.pallas import tpu_sc as plsc

N = 10000
E = 320000
D = 128
G = 64

NPAD = 10240
PAD_IDX = N
RNG = 320
ACC_ROWS = RNG + 8
EPT = E // 32
ECHUNK = 2000
NCH = EPT // ECHUNK
CAPP = (EPT // 128 + 6) * 128
BSTRIDE = 160
BLK = 512
NBLK = NPAD // BLK

_mesh = plsc.VectorSubcoreMesh(core_axis_name="c", subcore_axis_name="s")


@functools.partial(
    pl.kernel,
    out_type=[jax.ShapeDtypeStruct((32 * 32 * CAPP,), jnp.int32),
              jax.ShapeDtypeStruct((32 * 32 * CAPP,), jnp.int32),
              jax.ShapeDtypeStruct((32 * 512,), jnp.int32)],
    mesh=_mesh,
    scratch_types=[
        pltpu.VMEM((ECHUNK,), jnp.int32),
        pltpu.VMEM((ECHUNK,), jnp.int32),
        pltpu.VMEM((32 * BSTRIDE,), jnp.int32),
        pltpu.VMEM((32 * BSTRIDE,), jnp.int32),
        pltpu.VMEM((512,), jnp.int32),
        pltpu.SMEM((64,), jnp.int32),
    ],
)
def _part_sc(src_hbm, dst_hbm, zeros_hbm, selsrc_hbm, seldst_hbm, cnt_hbm,
             srcc_v, dstc_v, bsrc_v, bdst_v, cstage_v, sm):
    cc = lax.axis_index("c")
    ss = lax.axis_index("s")
    t = cc * 16 + ss
    iota = lax.iota(jnp.int32, 16)
    z16 = jnp.zeros((16,), jnp.int32)
    pltpu.sync_copy(zeros_hbm, bsrc_v)
    pltpu.sync_copy(zeros_hbm, bdst_v)
    for q in range(32):
        sm[q] = 0
        sm[32 + q] = 0

    def chunk_body(tt, carry):
        e0 = t * EPT + tt * ECHUNK
        pltpu.sync_copy(src_hbm.at[pl.ds(e0, ECHUNK)], srcc_v)
        pltpu.sync_copy(dst_hbm.at[pl.ds(e0, ECHUNK)], dstc_v)

        def vec_body(i, carry2):
            dvec = dstc_v[pl.ds(i * 16, 16)]
            svec = srcc_v[pl.ds(i * 16, 16)]
            qvec = (dvec * 6554) >> 21
            dlvec = dvec - qvec * RNG
            for l in range(16):
                q = qvec[l]
                s_ = svec[l]
                dl = dlvec[l]
                w = sm[q]
                lane = w & 15
                base = (w - lane) + q * BSTRIDE
                oh = iota == lane
                plsc.addupdate(bsrc_v.at[pl.ds(base, 16)],
                               jnp.where(oh, s_, 0))
                plsc.addupdate(bdst_v.at[pl.ds(base, 16)],
                               jnp.where(oh, dl, 0))
                w2 = w + 1

                @pl.when(w2 == 128)
                def _flush():
                    nb = sm[32 + q]
                    off = (q * 32 + t) * CAPP + nb * 128
                    pltpu.sync_copy(bsrc_v.at[pl.ds(q * BSTRIDE, 128)],
                                    selsrc_hbm.at[pl.ds(off, 128)])
                    pltpu.sync_copy(bdst_v.at[pl.ds(q * BSTRIDE, 128)],
                                    seldst_hbm.at[pl.ds(off, 128)])
                    for ii in range(8):
                        bsrc_v[pl.ds(q * BSTRIDE + ii * 16, 16)] = z16
                        bdst_v[pl.ds(q * BSTRIDE + ii * 16, 16)] = z16
                    sm[32 + q] = nb + 1

                sm[q] = w2 & 127
            return carry2

        lax.fori_loop(0, ECHUNK // 16, vec_body, 0)
        return carry

    lax.fori_loop(0, NCH, chunk_body, 0)

    for q in range(32):
        w = sm[q]
        nb = sm[32 + q]
        for ii in range(8):
            pos = iota + (ii * 16)
            sv = bsrc_v[pl.ds(q * BSTRIDE + ii * 16, 16)]
            dv = bdst_v[pl.ds(q * BSTRIDE + ii * 16, 16)]
            bsrc_v[pl.ds(q * BSTRIDE + ii * 16, 16)] = \
                jnp.where(pos >= w, PAD_IDX, sv)
            bdst_v[pl.ds(q * BSTRIDE + ii * 16, 16)] = \
                jnp.where(pos >= w, RNG, dv)
        off = (q * 32 + t) * CAPP + nb * 128
        pltpu.sync_copy(bsrc_v.at[pl.ds(q * BSTRIDE, 128)],
                        selsrc_hbm.at[pl.ds(off, 128)])
        pltpu.sync_copy(bdst_v.at[pl.ds(q * BSTRIDE, 128)],
                        seldst_hbm.at[pl.ds(off, 128)])
        for ii in range(8):
            bsrc_v[pl.ds(q * BSTRIDE + ii * 16, 16)] = z16 + PAD_IDX
            bdst_v[pl.ds(q * BSTRIDE + ii * 16, 16)] = z16 + RNG
        for jj in range(3):
            offj = off + (jj + 1) * 128
            pltpu.sync_copy(bsrc_v.at[pl.ds(q * BSTRIDE, 128)],
                            selsrc_hbm.at[pl.ds(offj, 128)])
            pltpu.sync_copy(bdst_v.at[pl.ds(q * BSTRIDE, 128)],
                            seldst_hbm.at[pl.ds(offj, 128)])
        cstage_v[pl.ds(q * 16, 16)] = z16 + (nb * 128 + w)
    pltpu.sync_copy(cstage_v, cnt_hbm.at[pl.ds(t * 512, 512)])


@functools.partial(
    pl.kernel,
    out_type=jax.ShapeDtypeStruct((NPAD * 16,), jnp.float32),
    mesh=_mesh,
    scratch_types=[
        pltpu.VMEM((128,), jnp.int32),
        pltpu.VMEM((16,), jnp.int32),
        pltpu.VMEM((ACC_ROWS * 16,), jnp.float32),
    ],
)
def _deg_sc(seldst_hbm, cnt_hbm, zeros_hbm, dp_hbm, dstl_v, cnt_v, acc_v):
    cc = lax.axis_index("c")
    ss = lax.axis_index("s")
    b = cc * 16 + ss
    pltpu.sync_copy(zeros_hbm, acc_v)
    ones = jnp.ones((16,), jnp.float32)

    def src_body(s, carry):
        pltpu.sync_copy(cnt_hbm.at[pl.ds(s * 512 + b * 16, 16)], cnt_v)
        cvec = cnt_v[pl.ds(0, 16)]
        nblk = (cvec[0] + 127) >> 7

        def blk_body(k, carry2):
            off = (b * 32 + s) * CAPP + k * 128
            pltpu.sync_copy(seldst_hbm.at[pl.ds(off, 128)], dstl_v)

            def grp_body(g, carry3):
                dvec = dstl_v[pl.ds(g * 16, 16)]
                for l in range(16):
                    d = dvec[l]
                    plsc.addupdate(acc_v.at[pl.ds(d * 16, 16)], ones)
                return carry3

            lax.fori_loop(0, 8, grp_body, 0)
            return carry2

        lax.fori_loop(0, nblk, blk_body, 0)
        return carry

    lax.fori_loop(0, 32, src_body, 0)
    pltpu.sync_copy(acc_v.at[pl.ds(0, RNG * 16)],
                    dp_hbm.at[pl.ds(b * RNG * 16, RNG * 16)])


@functools.partial(
    pl.kernel,
    out_type=jax.ShapeDtypeStruct((NPAD * D,), jnp.float32),
    mesh=_mesh,
    scratch_types=[
        pltpu.VMEM((128,), jnp.int32),
        pltpu.VMEM((128,), jnp.int32),
        pltpu.VMEM((128,), jnp.int32),
        pltpu.VMEM((128,), jnp.int32),
        pltpu.VMEM((16,), jnp.int32),
        pltpu.VMEM((128, D), jnp.float32),
        pltpu.VMEM((128, D), jnp.float32),
        pltpu.VMEM((ACC_ROWS * D,), jnp.float32),
        pltpu.SemaphoreType.DMA,
        pltpu.SemaphoreType.DMA,
    ],
)
def _agg_sc(hs_hbm, selsrc_hbm, seldst_hbm, cnt_hbm, zeros_hbm, out_hbm,
            src0_v, dstl0_v, src1_v, dstl1_v, cnt_v, rows0_v, rows1_v,
            acc_v, sem0, sem1):
    cc = lax.axis_index("c")
    ss = lax.axis_index("s")
    b = cc * 16 + ss
    pltpu.sync_copy(zeros_hbm, acc_v)

    def add_block(dstl_v, rows_v):
        def grp_body(g, carry3):
            dvec = dstl_v[pl.ds(g * 16, 16)]
            for l in range(16):
                d = dvec[l]
                base = d * D
                for kk in range(8):
                    val = rows_v[g * 16 + l, pl.ds(kk * 16, 16)]
                    plsc.addupdate(acc_v.at[pl.ds(base + kk * 16, 16)], val)
            return carry3

        lax.fori_loop(0, 8, grp_body, 0)

    def src_body(s, carry):
        pltpu.sync_copy(cnt_hbm.at[pl.ds(s * 512 + b * 16, 16)], cnt_v)
        cvec = cnt_v[pl.ds(0, 16)]
        nblk = (cvec[0] + 127) >> 7
        base0 = (b * 32 + s) * CAPP

        def pair_body(p, carry2):
            off0 = base0 + p * 256
            off1 = off0 + 128
            pltpu.sync_copy(selsrc_hbm.at[pl.ds(off0, 128)], src0_v)
            pltpu.sync_copy(seldst_hbm.at[pl.ds(off0, 128)], dstl0_v)
            cp0 = pltpu.async_copy(hs_hbm.at[src0_v], rows0_v, sem0)
            pltpu.sync_copy(selsrc_hbm.at[pl.ds(off1, 128)], src1_v)
            pltpu.sync_copy(seldst_hbm.at[pl.ds(off1, 128)], dstl1_v)
            cp1 = pltpu.async_copy(hs_hbm.at[src1_v], rows1_v, sem1)
            cp0.wait()
            add_block(dstl0_v, rows0_v)
            cp1.wait()
            add_block(dstl1_v, rows1_v)
            return carry2

        lax.fori_loop(0, nblk >> 1, pair_body, 0)

        @pl.when((nblk & 1) == 1)
        def _tail():
            off = base0 + (nblk - 1) * 128
            pltpu.sync_copy(selsrc_hbm.at[pl.ds(off, 128)], src0_v)
            pltpu.sync_copy(seldst_hbm.at[pl.ds(off, 128)], dstl0_v)
            pltpu.async_copy(hs_hbm.at[src0_v], rows0_v, sem0).wait()
            add_block(dstl0_v, rows0_v)

        return carry

    lax.fori_loop(0, 32, src_body, 0)
    pltpu.sync_copy(acc_v.at[pl.ds(0, RNG * D)],
                    out_hbm.at[pl.ds(b * RNG * D, RNG * D)])


def _dinv_block(dp_ref):
    deg = jnp.sum(dp_ref[...], axis=1, keepdims=True) * 0.0625 + 1.0
    return lax.rsqrt(deg)


def _tc1_body(x_ref, w_ref, dp_ref, hs_ref):
    dinv = _dinv_block(dp_ref)
    xw = jnp.dot(x_ref[...], w_ref[...], preferred_element_type=jnp.float32)
    hs_ref[...] = xw * dinv


def _tc2_body(hs_ref, p_ref, dp_ref, b_ref, w_ref, out_ref):
    dinv = _dinv_block(dp_ref)
    tot = hs_ref[...] + p_ref[...]
    h = jnp.maximum(tot * dinv + b_ref[...], 0.0)
    out_ref[...] = (
        jnp.dot(h, w_ref[...], preferred_element_type=jnp.float32) * dinv
    )


def _tc3_body(hs_ref, p_ref, dp_ref, b_ref, batch_ref, sums_ref, cnts_ref):
    i = pl.program_id(0)
    dinv = _dinv_block(dp_ref)
    h3 = (hs_ref[...] + p_ref[...]) * dinv + b_ref[...]
    gids = lax.broadcasted_iota(jnp.int32, (1, G), 1)
    onehot = (batch_ref[...] == gids).astype(jnp.float32)
    dn = (((0,), (0,)), ((), ()))
    sm = lax.dot_general(onehot, h3, dn, preferred_element_type=jnp.float32)
    ones = jnp.ones((BLK, D), jnp.float32)
    cn = lax.dot_general(onehot, ones, dn, preferred_element_type=jnp.float32)

    @pl.when(i == 0)
    def _init():
        sums_ref[...] = sm
        cnts_ref[...] = cn

    @pl.when(i > 0)
    def _acc():
        sums_ref[...] += sm
        cnts_ref[...] += cn


def _tc4_body(sums_ref, cnts_ref, w1, b1, w2, b2, w3, b3, w4, b4, f_ref,
              y_ref):
    f = sums_ref[...] / jnp.maximum(cnts_ref[...], 1.0)
    f_ref[...] = f
    y = jnp.maximum(jnp.dot(f, w1[...], preferred_element_type=jnp.float32)
                    + b1[...], 0.0)
    y = jnp.maximum(jnp.dot(y, w2[...], preferred_element_type=jnp.float32)
                    + b2[...], 0.0)
    y = jnp.maximum(jnp.dot(y, w3[...], preferred_element_type=jnp.float32)
                    + b3[...], 0.0)
    y_ref[...] = jnp.dot(y, w4[...], preferred_element_type=jnp.float32) \
        + b4[...]


def _full(shape):
    return pl.BlockSpec(shape, lambda i: (0,) * len(shape))


_BS_ROWS = pl.BlockSpec((BLK, D), lambda i: (i, 0))
_BS_DP = pl.BlockSpec((BLK, 16), lambda i: (i, 0))


def _tc1(x_pad, W1, dp):
    return pl.pallas_call(
        _tc1_body,
        grid=(NBLK,),
        in_specs=[_BS_ROWS, _full((D, D)), _BS_DP],
        out_specs=_BS_ROWS,
        out_shape=jax.ShapeDtypeStruct((NPAD, D), jnp.float32),
    )(x_pad, W1, dp)


def _tc2(hs, p, dp, b, Wn):
    return pl.pallas_call(
        _tc2_body,
        grid=(NBLK,),
        in_specs=[_BS_ROWS, _BS_ROWS, _BS_DP, _full((1, D)), _full((D, D))],
        out_specs=_BS_ROWS,
        out_shape=jax.ShapeDtypeStruct((NPAD, D), jnp.float32),
    )(hs, p, dp, b, Wn)


def _tc3(hs, p, dp, b, batch_pad):
    return pl.pallas_call(
        _tc3_body,
        grid=(NBLK,),
        in_specs=[_BS_ROWS, _BS_ROWS, _BS_DP, _full((1, D)),
                  pl.BlockSpec((BLK, 1), lambda i: (i, 0))],
        out_specs=[_full((G, D)), _full((G, D))],
        out_shape=[jax.ShapeDtypeStruct((G, D), jnp.float32),
                   jax.ShapeDtypeStruct((G, D), jnp.float32)],
    )(hs, p, dp, b, batch_pad)


def _tc4(sums, cnts, fcW1, fcb1, fcW2, fcb2, fcW3, fcb3, fcW4, fcb4):
    return pl.pallas_call(
        _tc4_body,
        out_shape=[jax.ShapeDtypeStruct((G, D), jnp.float32),
                   jax.ShapeDtypeStruct((G, 32), jnp.float32)],
    )(sums, cnts, fcW1, fcb1, fcW2, fcb2, fcW3, fcb3, fcW4, fcb4)


def kernel(x, edge_index, batch, W1, b1, W2, b2, W3, b3, fcW1, fcb1, fcW2,
           fcb2, fcW3, fcb3, fcW4, fcb4):
    x_pad = jnp.zeros((NPAD, D), jnp.float32).at[:N].set(x)
    src_flat = edge_index[0]
    dst_flat = edge_index[1]
    batch_pad = jnp.concatenate(
        [batch, jnp.full((NPAD - N,), G, jnp.int32)]).reshape(NPAD, 1)
    zeros_i = jnp.zeros((32 * BSTRIDE,), jnp.int32)
    zeros16 = jnp.zeros((ACC_ROWS * 16,), jnp.float32)
    zerosD = jnp.zeros((ACC_ROWS * D,), jnp.float32)

    selsrc, seldst, cnt = _part_sc(src_flat, dst_flat, zeros_i)
    dp = _deg_sc(seldst, cnt, zeros16).reshape(NPAD, 16)

    hs1 = _tc1(x_pad, W1, dp)
    p1 = _agg_sc(hs1, selsrc, seldst, cnt, zerosD).reshape(NPAD, D)
    hs2 = _tc2(hs1, p1, dp, b1.reshape(1, D), W2)
    p2 = _agg_sc(hs2, selsrc, seldst, cnt, zerosD).reshape(NPAD, D)
    hs3 = _tc2(hs2, p2, dp, b2.reshape(1, D), W3)
    p3 = _agg_sc(hs3, selsrc, seldst, cnt, zerosD).reshape(NPAD, D)
    sums, cnts = _tc3(hs3, p3, dp, b3.reshape(1, D), batch_pad)
    f, y = _tc4(sums, cnts, fcW1, fcb1.reshape(1, 4 * D), fcW2,
                fcb2.reshape(1, 2 * D), fcW3, fcb3.reshape(1, D), fcW4,
                fcb4.reshape(1, 32))
    return (f, y)


# speedup vs baseline: 3.0798x; 1.0002x over previous
import functools

import jax
import jax.numpy as jnp
from jax import lax
from jax.experimental import pallas as pl
from jax.experimental.pallas import tpu as pltpu
from jax.experimental.pallas import tpu_sc as plsc

N = 10000
E = 320000
D = 128
G = 64

NPAD = 10240
PAD_IDX = N
RNG = 320
ACC_ROWS = RNG + 8
EPT = E // 32
ECHUNK = 2000
NCH = EPT // ECHUNK
CAPP = (EPT // 128 + 6) * 128
BSTRIDE = 160
BLK = 512
NBLK = NPAD // BLK

_mesh = plsc.VectorSubcoreMesh(core_axis_name="c", subcore_axis_name="s")


@functools.partial(
    pl.kernel,
    out_type=[jax.ShapeDtypeStruct((32 * 32 * CAPP,), jnp.int32),
              jax.ShapeDtypeStruct((32 * 32 * CAPP,), jnp.int32),
              jax.ShapeDtypeStruct((32 * 512,), jnp.int32)],
    mesh=_mesh,
    scratch_types=[
        pltpu.VMEM((ECHUNK,), jnp.int32),
        pltpu.VMEM((ECHUNK,), jnp.int32),
        pltpu.VMEM((32 * BSTRIDE,), jnp.int32),
        pltpu.VMEM((32 * BSTRIDE,), jnp.int32),
        pltpu.VMEM((512,), jnp.int32),
        pltpu.SMEM((64,), jnp.int32),
    ],
)
def _part_sc(src_hbm, dst_hbm, zeros_hbm, selsrc_hbm, seldst_hbm, cnt_hbm,
             srcc_v, dstc_v, bsrc_v, bdst_v, cstage_v, sm):
    cc = lax.axis_index("c")
    ss = lax.axis_index("s")
    t = cc * 16 + ss
    iota = lax.iota(jnp.int32, 16)
    z16 = jnp.zeros((16,), jnp.int32)
    pltpu.sync_copy(zeros_hbm, bsrc_v)
    pltpu.sync_copy(zeros_hbm, bdst_v)
    for q in range(32):
        sm[q] = 0
        sm[32 + q] = 0

    def chunk_body(tt, carry):
        e0 = t * EPT + tt * ECHUNK
        pltpu.sync_copy(src_hbm.at[pl.ds(e0, ECHUNK)], srcc_v)
        pltpu.sync_copy(dst_hbm.at[pl.ds(e0, ECHUNK)], dstc_v)

        def vec_body(i, carry2):
            dvec = dstc_v[pl.ds(i * 16, 16)]
            svec = srcc_v[pl.ds(i * 16, 16)]
            qvec = (dvec * 6554) >> 21
            dlvec = dvec - qvec * RNG
            for l in range(16):
                q = qvec[l]
                s_ = svec[l]
                dl = dlvec[l]
                w = sm[q]
                lane = w & 15
                base = (w - lane) + q * BSTRIDE
                oh = iota == lane
                plsc.addupdate(bsrc_v.at[pl.ds(base, 16)],
                               jnp.where(oh, s_, 0))
                plsc.addupdate(bdst_v.at[pl.ds(base, 16)],
                               jnp.where(oh, dl, 0))
                w2 = w + 1

                @pl.when(w2 == 128)
                def _flush():
                    nb = sm[32 + q]
                    off = (q * 32 + t) * CAPP + nb * 128
                    pltpu.sync_copy(bsrc_v.at[pl.ds(q * BSTRIDE, 128)],
                                    selsrc_hbm.at[pl.ds(off, 128)])
                    pltpu.sync_copy(bdst_v.at[pl.ds(q * BSTRIDE, 128)],
                                    seldst_hbm.at[pl.ds(off, 128)])
                    for ii in range(8):
                        bsrc_v[pl.ds(q * BSTRIDE + ii * 16, 16)] = z16
                        bdst_v[pl.ds(q * BSTRIDE + ii * 16, 16)] = z16
                    sm[32 + q] = nb + 1

                sm[q] = w2 & 127
            return carry2

        lax.fori_loop(0, ECHUNK // 16, vec_body, 0)
        return carry

    lax.fori_loop(0, NCH, chunk_body, 0)

    for q in range(32):
        w = sm[q]
        nb = sm[32 + q]
        for ii in range(8):
            pos = iota + (ii * 16)
            sv = bsrc_v[pl.ds(q * BSTRIDE + ii * 16, 16)]
            dv = bdst_v[pl.ds(q * BSTRIDE + ii * 16, 16)]
            bsrc_v[pl.ds(q * BSTRIDE + ii * 16, 16)] = \
                jnp.where(pos >= w, PAD_IDX, sv)
            bdst_v[pl.ds(q * BSTRIDE + ii * 16, 16)] = \
                jnp.where(pos >= w, RNG, dv)
        off = (q * 32 + t) * CAPP + nb * 128
        pltpu.sync_copy(bsrc_v.at[pl.ds(q * BSTRIDE, 128)],
                        selsrc_hbm.at[pl.ds(off, 128)])
        pltpu.sync_copy(bdst_v.at[pl.ds(q * BSTRIDE, 128)],
                        seldst_hbm.at[pl.ds(off, 128)])
        for ii in range(8):
            bsrc_v[pl.ds(q * BSTRIDE + ii * 16, 16)] = z16 + PAD_IDX
            bdst_v[pl.ds(q * BSTRIDE + ii * 16, 16)] = z16 + RNG
        for jj in range(3):
            offj = off + (jj + 1) * 128
            pltpu.sync_copy(bsrc_v.at[pl.ds(q * BSTRIDE, 128)],
                            selsrc_hbm.at[pl.ds(offj, 128)])
            pltpu.sync_copy(bdst_v.at[pl.ds(q * BSTRIDE, 128)],
                            seldst_hbm.at[pl.ds(offj, 128)])
        cstage_v[pl.ds(q * 16, 16)] = z16 + (nb * 128 + w)
    pltpu.sync_copy(cstage_v, cnt_hbm.at[pl.ds(t * 512, 512)])


@functools.partial(
    pl.kernel,
    out_type=jax.ShapeDtypeStruct((NPAD * 16,), jnp.float32),
    mesh=_mesh,
    scratch_types=[
        pltpu.VMEM((128,), jnp.int32),
        pltpu.VMEM((16,), jnp.int32),
        pltpu.VMEM((ACC_ROWS * 16,), jnp.float32),
    ],
)
def _deg_sc(seldst_hbm, cnt_hbm, zeros_hbm, dp_hbm, dstl_v, cnt_v, acc_v):
    cc = lax.axis_index("c")
    ss = lax.axis_index("s")
    b = cc * 16 + ss
    pltpu.sync_copy(zeros_hbm, acc_v)
    ones = jnp.ones((16,), jnp.float32)

    def src_body(s, carry):
        pltpu.sync_copy(cnt_hbm.at[pl.ds(s * 512 + b * 16, 16)], cnt_v)
        cvec = cnt_v[pl.ds(0, 16)]
        nblk = (cvec[0] + 127) >> 7

        def blk_body(k, carry2):
            off = (b * 32 + s) * CAPP + k * 128
            pltpu.sync_copy(seldst_hbm.at[pl.ds(off, 128)], dstl_v)

            def grp_body(g, carry3):
                dvec = dstl_v[pl.ds(g * 16, 16)]
                for l in range(16):
                    d = dvec[l]
                    plsc.addupdate(acc_v.at[pl.ds(d * 16, 16)], ones)
                return carry3

            lax.fori_loop(0, 8, grp_body, 0)
            return carry2

        lax.fori_loop(0, nblk, blk_body, 0)
        return carry

    lax.fori_loop(0, 32, src_body, 0)
    pltpu.sync_copy(acc_v.at[pl.ds(0, RNG * 16)],
                    dp_hbm.at[pl.ds(b * RNG * 16, RNG * 16)])


@functools.partial(
    pl.kernel,
    out_type=jax.ShapeDtypeStruct((NPAD * D,), jnp.float32),
    mesh=_mesh,
    scratch_types=[
        pltpu.VMEM((128,), jnp.int32),
        pltpu.VMEM((128,), jnp.int32),
        pltpu.VMEM((16,), jnp.int32),
        pltpu.VMEM((128, D), jnp.float32),
        pltpu.VMEM((ACC_ROWS * D,), jnp.float32),
        pltpu.SemaphoreType.DMA,
    ],
)
def _agg_sc(hs_hbm, selsrc_hbm, seldst_hbm, cnt_hbm, zeros_hbm, out_hbm,
            src_v, dstl_v, cnt_v, rows_v, acc_v, sem):
    cc = lax.axis_index("c")
    ss = lax.axis_index("s")
    b = cc * 16 + ss
    pltpu.sync_copy(zeros_hbm, acc_v)

    def src_body(s, carry):
        pltpu.sync_copy(cnt_hbm.at[pl.ds(s * 512 + b * 16, 16)], cnt_v)
        cvec = cnt_v[pl.ds(0, 16)]
        nblk = (cvec[0] + 127) >> 7

        def blk_body(k, carry2):
            off = (b * 32 + s) * CAPP + k * 128
            pltpu.sync_copy(selsrc_hbm.at[pl.ds(off, 128)], src_v)
            pltpu.sync_copy(seldst_hbm.at[pl.ds(off, 128)], dstl_v)
            pltpu.async_copy(hs_hbm.at[src_v], rows_v, sem).wait()

            def grp_body(g, carry3):
                dvec = dstl_v[pl.ds(g * 16, 16)]
                for l in range(16):
                    d = dvec[l]
                    base = d * D
                    for kk in range(8):
                        val = rows_v[g * 16 + l, pl.ds(kk * 16, 16)]
                        plsc.addupdate(acc_v.at[pl.ds(base + kk * 16, 16)],
                                       val)
                return carry3

            lax.fori_loop(0, 8, grp_body, 0)
            return carry2

        lax.fori_loop(0, nblk, blk_body, 0)
        return carry

    lax.fori_loop(0, 32, src_body, 0)
    pltpu.sync_copy(acc_v.at[pl.ds(0, RNG * D)],
                    out_hbm.at[pl.ds(b * RNG * D, RNG * D)])


def _dinv_block(dp_ref):
    deg = jnp.sum(dp_ref[...], axis=1, keepdims=True) * 0.0625 + 1.0
    return lax.rsqrt(deg)


def _tc1_body(x_ref, w_ref, dp_ref, hs_ref):
    dinv = _dinv_block(dp_ref)
    xw = jnp.dot(x_ref[...], w_ref[...], preferred_element_type=jnp.float32)
    hs_ref[...] = xw * dinv


def _tc2_body(hs_ref, p_ref, dp_ref, b_ref, w_ref, out_ref):
    dinv = _dinv_block(dp_ref)
    tot = hs_ref[...] + p_ref[...]
    h = jnp.maximum(tot * dinv + b_ref[...], 0.0)
    out_ref[...] = (
        jnp.dot(h, w_ref[...], preferred_element_type=jnp.float32) * dinv
    )


def _tc3_body(hs_ref, p_ref, dp_ref, b_ref, batch_ref, sums_ref, cnts_ref):
    i = pl.program_id(0)
    dinv = _dinv_block(dp_ref)
    h3 = (hs_ref[...] + p_ref[...]) * dinv + b_ref[...]
    gids = lax.broadcasted_iota(jnp.int32, (1, G), 1)
    onehot = (batch_ref[...] == gids).astype(jnp.float32)
    dn = (((0,), (0,)), ((), ()))
    sm = lax.dot_general(onehot, h3, dn, preferred_element_type=jnp.float32)
    ones = jnp.ones((BLK, D), jnp.float32)
    cn = lax.dot_general(onehot, ones, dn, preferred_element_type=jnp.float32)

    @pl.when(i == 0)
    def _init():
        sums_ref[...] = sm
        cnts_ref[...] = cn

    @pl.when(i > 0)
    def _acc():
        sums_ref[...] += sm
        cnts_ref[...] += cn


def _tc4_body(sums_ref, cnts_ref, w1, b1, w2, b2, w3, b3, w4, b4, f_ref,
              y_ref):
    f = sums_ref[...] / jnp.maximum(cnts_ref[...], 1.0)
    f_ref[...] = f
    y = jnp.maximum(jnp.dot(f, w1[...], preferred_element_type=jnp.float32)
                    + b1[...], 0.0)
    y = jnp.maximum(jnp.dot(y, w2[...], preferred_element_type=jnp.float32)
                    + b2[...], 0.0)
    y = jnp.maximum(jnp.dot(y, w3[...], preferred_element_type=jnp.float32)
                    + b3[...], 0.0)
    y_ref[...] = jnp.dot(y, w4[...], preferred_element_type=jnp.float32) \
        + b4[...]


def _full(shape):
    return pl.BlockSpec(shape, lambda i: (0,) * len(shape))


_BS_ROWS = pl.BlockSpec((BLK, D), lambda i: (i, 0))
_BS_DP = pl.BlockSpec((BLK, 16), lambda i: (i, 0))


def _tc1(x_pad, W1, dp):
    return pl.pallas_call(
        _tc1_body,
        grid=(NBLK,),
        in_specs=[_BS_ROWS, _full((D, D)), _BS_DP],
        out_specs=_BS_ROWS,
        out_shape=jax.ShapeDtypeStruct((NPAD, D), jnp.float32),
    )(x_pad, W1, dp)


def _tc2(hs, p, dp, b, Wn):
    return pl.pallas_call(
        _tc2_body,
        grid=(NBLK,),
        in_specs=[_BS_ROWS, _BS_ROWS, _BS_DP, _full((1, D)), _full((D, D))],
        out_specs=_BS_ROWS,
        out_shape=jax.ShapeDtypeStruct((NPAD, D), jnp.float32),
    )(hs, p, dp, b, Wn)


def _tc3(hs, p, dp, b, batch_pad):
    return pl.pallas_call(
        _tc3_body,
        grid=(NBLK,),
        in_specs=[_BS_ROWS, _BS_ROWS, _BS_DP, _full((1, D)),
                  pl.BlockSpec((BLK, 1), lambda i: (i, 0))],
        out_specs=[_full((G, D)), _full((G, D))],
        out_shape=[jax.ShapeDtypeStruct((G, D), jnp.float32),
                   jax.ShapeDtypeStruct((G, D), jnp.float32)],
    )(hs, p, dp, b, batch_pad)


def _tc4(sums, cnts, fcW1, fcb1, fcW2, fcb2, fcW3, fcb3, fcW4, fcb4):
    return pl.pallas_call(
        _tc4_body,
        out_shape=[jax.ShapeDtypeStruct((G, D), jnp.float32),
                   jax.ShapeDtypeStruct((G, 32), jnp.float32)],
    )(sums, cnts, fcW1, fcb1, fcW2, fcb2, fcW3, fcb3, fcW4, fcb4)


def kernel(x, edge_index, batch, W1, b1, W2, b2, W3, b3, fcW1, fcb1, fcW2,
           fcb2, fcW3, fcb3, fcW4, fcb4):
    x_pad = jnp.zeros((NPAD, D), jnp.float32).at[:N].set(x)
    src_flat = edge_index[0]
    dst_flat = edge_index[1]
    batch_pad = jnp.concatenate(
        [batch, jnp.full((NPAD - N,), G, jnp.int32)]).reshape(NPAD, 1)
    zeros_i = jnp.zeros((32 * BSTRIDE,), jnp.int32)
    zeros16 = jnp.zeros((ACC_ROWS * 16,), jnp.float32)
    zerosD = jnp.zeros((ACC_ROWS * D,), jnp.float32)

    selsrc, seldst, cnt = _part_sc(src_flat, dst_flat, zeros_i)
    dp = _deg_sc(seldst, cnt, zeros16).reshape(NPAD, 16)

    hs1 = _tc1(x_pad, W1, dp)
    p1 = _agg_sc(hs1, selsrc, seldst, cnt, zerosD).reshape(NPAD, D)
    hs2 = _tc2(hs1, p1, dp, b1.reshape(1, D), W2)
    p2 = _agg_sc(hs2, selsrc, seldst, cnt, zerosD).reshape(NPAD, D)
    hs3 = _tc2(hs2, p2, dp, b2.reshape(1, D), W3)
    p3 = _agg_sc(hs3, selsrc, seldst, cnt, zerosD).reshape(NPAD, D)
    sums, cnts = _tc3(hs3, p3, dp, b3.reshape(1, D), batch_pad)
    f, y = _tc4(sums, cnts, fcW1, fcb1.reshape(1, 4 * D), fcW2,
                fcb2.reshape(1, 2 * D), fcW3, fcb3.reshape(1, D), fcW4,
                fcb4.reshape(1, 32))
    return (f, y)
